# baseline (speedup 1.0000x reference)
import functools

import numpy as np
import jax
import jax.numpy as jnp
from jax import lax
from jax.experimental import pallas as pl
from jax.experimental.pallas import tpu as pltpu
from jax.experimental.pallas import tpu_sc as plsc

_MAXLEN = 2048
_D = 128
_B = 4
_BT = _B * _MAXLEN
_NC, _NS, _L = 2, 16, 16
_NW = _NC * _NS
_LPW = _MAXLEN // _NW
_NPH = 4
_SL = _LPW // _NPH


def _positional_encoding():
    pos = np.arange(_MAXLEN)[:, np.newaxis]
    i = np.arange(_D)[np.newaxis, :]
    angle = pos * (1.0 / np.power(10000, 2 * (i // 2) / np.float32(_D)))
    angle[:, 0::2] = np.sin(angle[:, 0::2])
    angle[:, 1::2] = np.cos(angle[:, 1::2])
    return angle.astype(np.float32)


_POS = _positional_encoding()

_mesh = plsc.VectorSubcoreMesh(core_axis_name="c", subcore_axis_name="s")


@functools.partial(
    pl.kernel,
    mesh=_mesh,
    out_type=jax.ShapeDtypeStruct((_BT, _D), jnp.float32),
    scratch_types=[
        pltpu.VMEM((_B, _LPW), jnp.int32),
        pltpu.VMEM((_LPW, _D), jnp.float32),
        pltpu.VMEM((_B * _LPW, _D), jnp.float32),
        pltpu.SemaphoreType.DMA,
        [pltpu.SemaphoreType.DMA] * (_B * _NPH),
        pltpu.SemaphoreType.DMA,
    ],
)
def _emb_kernel(x_hbm, table_hbm, pos_hbm, out_hbm, idx_v, pos_v, rows_v,
                psem, gsems, osem):
    wid = lax.axis_index("s") * _NC + lax.axis_index("c")
    l0 = wid * _LPW

    def gather(k, b):
        return pltpu.async_copy(
            table_hbm.at[idx_v.at[b, pl.ds(k * _SL, _SL)]],
            rows_v.at[pl.ds(b * _LPW + k * _SL, _SL)],
            gsems[k * _B + b],
        )

    pltpu.sync_copy(x_hbm.at[:, wid], idx_v)
    g_h = [gather(0, 0)]
    ph = pltpu.async_copy(pos_hbm.at[pl.ds(l0, _LPW)], pos_v, psem)
    for k in range(_NPH):
        for b in range(_B):
            if k == 0 and b == 0:
                continue
            g_h.append(gather(k, b))
    ph.wait()

    o_h = []
    for k in range(_NPH):
        for b in range(_B):
            g_h[k * _B + b].wait()

        def add_pos(p, carry, k=k):
            i = k * _SL + p
            pv = [pos_v[i, pl.ds(j * _L, _L)] for j in range(_D // _L)]
            for b in range(_B):
                r = b * _LPW + i
                for j in range(_D // _L):
                    s = pl.ds(j * _L, _L)
                    rows_v[r, s] = rows_v[r, s] + pv[j]
            return carry

        lax.fori_loop(0, _SL, add_pos, 0)
        for b in range(_B):
            o_h.append(
                pltpu.async_copy(
                    rows_v.at[pl.ds(b * _LPW + k * _SL, _SL)],
                    out_hbm.at[pl.ds(b * _MAXLEN + l0 + k * _SL, _SL)],
                    osem,
                )
            )
    for h in o_h:
        h.wait()


def kernel(x, table):
    idx = x.reshape(_B, _NW, _LPW).astype(jnp.int32)
    out = _emb_kernel(idx, table, jnp.asarray(_POS))
    return out.reshape(_B, _MAXLEN, _D)


# speedup vs baseline: 1.0245x; 1.0245x over previous
import functools

import numpy as np
import jax
import jax.numpy as jnp
from jax import lax
from jax.experimental import pallas as pl
from jax.experimental.pallas import tpu as pltpu
from jax.experimental.pallas import tpu_sc as plsc

_MAXLEN = 2048
_D = 128
_B = 4
_BT = _B * _MAXLEN
_NC, _NS, _L = 2, 16, 16
_NW = _NC * _NS
_LPW = _MAXLEN // _NW
_NPH = 2
_SL = _LPW // _NPH


def _positional_encoding():
    pos = np.arange(_MAXLEN)[:, np.newaxis]
    i = np.arange(_D)[np.newaxis, :]
    angle = pos * (1.0 / np.power(10000, 2 * (i // 2) / np.float32(_D)))
    angle[:, 0::2] = np.sin(angle[:, 0::2])
    angle[:, 1::2] = np.cos(angle[:, 1::2])
    return angle.astype(np.float32)


_POS = _positional_encoding()

_mesh = plsc.VectorSubcoreMesh(core_axis_name="c", subcore_axis_name="s")


@functools.partial(
    pl.kernel,
    mesh=_mesh,
    out_type=jax.ShapeDtypeStruct((_BT, _D), jnp.float32),
    scratch_types=[
        pltpu.VMEM((_B, _LPW), jnp.int32),
        pltpu.VMEM((_LPW, _D), jnp.float32),
        pltpu.VMEM((_B * _LPW, _D), jnp.float32),
        pltpu.SemaphoreType.DMA,
        [pltpu.SemaphoreType.DMA] * (_B * _NPH),
        pltpu.SemaphoreType.DMA,
    ],
)
def _emb_kernel(x_hbm, table_hbm, pos_hbm, out_hbm, idx_v, pos_v, rows_v,
                psem, gsems, osem):
    wid = lax.axis_index("s") * _NC + lax.axis_index("c")
    l0 = wid * _LPW

    def gather(k, b):
        return pltpu.async_copy(
            table_hbm.at[idx_v.at[b, pl.ds(k * _SL, _SL)]],
            rows_v.at[pl.ds(b * _LPW + k * _SL, _SL)],
            gsems[k * _B + b],
        )

    pltpu.sync_copy(x_hbm.at[:, wid], idx_v)
    g_h = [gather(0, 0)]
    ph = pltpu.async_copy(pos_hbm.at[pl.ds(l0, _LPW)], pos_v, psem)
    for k in range(_NPH):
        for b in range(_B):
            if k == 0 and b == 0:
                continue
            g_h.append(gather(k, b))
    ph.wait()

    o_h = []
    for k in range(_NPH):
        for b in range(_B):
            g_h[k * _B + b].wait()
        subs = [(0, _SL)] if k < _NPH - 1 else [(0, _SL // 2), (_SL // 2, _SL)]
        for lo, hi in subs:

            def add_pos(p, carry, k=k):
                i = k * _SL + p
                pv = [pos_v[i, pl.ds(j * _L, _L)] for j in range(_D // _L)]
                for b in range(_B):
                    r = b * _LPW + i
                    for j in range(_D // _L):
                        s = pl.ds(j * _L, _L)
                        rows_v[r, s] = rows_v[r, s] + pv[j]
                return carry

            lax.fori_loop(lo, hi, add_pos, 0)
            for b in range(_B):
                o_h.append(
                    pltpu.async_copy(
                        rows_v.at[pl.ds(b * _LPW + k * _SL + lo, hi - lo)],
                        out_hbm.at[pl.ds(b * _MAXLEN + l0 + k * _SL + lo, hi - lo)],
                        osem,
                    )
                )
    for h in o_h:
        h.wait()


def kernel(x, table):
    idx = x.reshape(_B, _NW, _LPW).astype(jnp.int32)
    out = _emb_kernel(idx, table, jnp.asarray(_POS))
    return out.reshape(_B, _MAXLEN, _D)
